# baseline (speedup 1.0000x reference)
import functools

import jax
import jax.numpy as jnp
from jax import lax
from jax.experimental import pallas as pl
from jax.experimental.pallas import tpu as pltpu
from jax.experimental.pallas import tpu_sc as plsc

VOCAB = 1000000
EMBED_DIM = 64
LANES = 128
VREG = 16
BATCH = 4096
SEQ_LEN = 200

N = BATCH * SEQ_LEN
CHUNK = 200
UNROLL = 8


def _make_sc_gather():
    info = plsc.get_sparse_core_info()
    nc, ns = info.num_cores, info.num_subcores
    nw = nc * ns
    per_w = N // nw
    chunks_per_w = per_w // CHUNK
    npairs = chunks_per_w // 2

    mesh = plsc.VectorSubcoreMesh(core_axis_name="c", subcore_axis_name="s")

    @functools.partial(
        pl.kernel,
        mesh=mesh,
        out_type=jax.ShapeDtypeStruct((N, EMBED_DIM), jnp.float32),
        scratch_types=[
            pltpu.VMEM((per_w,), jnp.int32),
            pltpu.VMEM((2, CHUNK, LANES), jnp.float32),
            pltpu.VMEM((2, CHUNK, EMBED_DIM), jnp.float32),
            pltpu.SemaphoreType.DMA,
            pltpu.SemaphoreType.DMA,
            pltpu.SemaphoreType.DMA,
            pltpu.SemaphoreType.DMA,
        ],
    )
    def k(idx_hbm, table_hbm, out_hbm, idx_v, g_v, w_v, g0, g1, w0, w1):
        wid = lax.axis_index("s") * nc + lax.axis_index("c")
        base = wid * per_w
        gsem = (g0, g1)
        wsem = (w0, w1)
        pltpu.sync_copy(idx_hbm.at[pl.ds(base, per_w)], idx_v)

        def issue_gather(c, b):
            pltpu.async_copy(
                table_hbm.at[idx_v.at[pl.ds(c * CHUNK, CHUNK)]],
                g_v.at[b],
                gsem[b],
            )

        def wait_gather(b):
            pltpu.make_async_copy(
                table_hbm.at[pl.ds(0, CHUNK)], g_v.at[b], gsem[b]
            ).wait()

        def issue_wb(c, b):
            pltpu.async_copy(
                w_v.at[b],
                out_hbm.at[pl.ds(base + c * CHUNK, CHUNK)],
                wsem[b],
            )

        def wait_wb(b):
            pltpu.make_async_copy(
                w_v.at[b],
                out_hbm.at[pl.ds(base, CHUNK)],
                wsem[b],
            ).wait()

        def compact(b):
            gb = g_v.at[b]
            wb = w_v.at[b]

            def rows(i, carry):
                for u in range(UNROLL):
                    r = i * UNROLL + u
                    for l in range(EMBED_DIM // VREG):
                        wb[r, pl.ds(l * VREG, VREG)] = gb[r, pl.ds(l * VREG, VREG)]
                return carry

            lax.fori_loop(0, CHUNK // UNROLL, rows, 0)

        issue_gather(0, 0)
        issue_gather(1, 1)

        def body(p, carry):
            for b in (0, 1):
                c = 2 * p + b
                wait_gather(b)

                @pl.when(p > 0)
                def _():
                    wait_wb(b)

                compact(b)

                @pl.when(p < npairs - 1)
                def _():
                    issue_gather(c + 2, b)

                issue_wb(c, b)
            return carry

        lax.fori_loop(0, npairs, body, 0)
        wait_wb(0)
        wait_wb(1)

    return k


def kernel(batch, table):
    k = _make_sc_gather()
    table128 = jnp.pad(table, ((0, 0), (0, LANES - EMBED_DIM)))
    idx = batch.reshape(N)
    out = k(idx, table128)
    return out.reshape(BATCH, SEQ_LEN, EMBED_DIM)


# speedup vs baseline: 1.0038x; 1.0038x over previous
import functools

import jax
import jax.numpy as jnp
from jax import lax
from jax.experimental import pallas as pl
from jax.experimental.pallas import tpu as pltpu
from jax.experimental.pallas import tpu_sc as plsc

VOCAB = 1000000
EMBED_DIM = 64
LANES = 128
VREG = 16
BATCH = 4096
SEQ_LEN = 200

N = BATCH * SEQ_LEN
CHUNK = 200
UNROLL = 8


def _make_sc_gather():
    info = plsc.get_sparse_core_info()
    nc, ns = info.num_cores, info.num_subcores
    nw = nc * ns
    per_w = N // nw
    chunks_per_w = per_w // CHUNK
    npairs = chunks_per_w // 2

    mesh = plsc.VectorSubcoreMesh(core_axis_name="c", subcore_axis_name="s")

    @functools.partial(
        pl.kernel,
        mesh=mesh,
        out_type=jax.ShapeDtypeStruct((N, EMBED_DIM), jnp.float32),
        scratch_types=[
            pltpu.VMEM((per_w,), jnp.int32),
            pltpu.VMEM((2, CHUNK, LANES), jnp.float32),
            pltpu.VMEM((2, CHUNK, EMBED_DIM), jnp.float32),
            pltpu.SemaphoreType.DMA,
            pltpu.SemaphoreType.DMA,
            pltpu.SemaphoreType.DMA,
            pltpu.SemaphoreType.DMA,
        ],
    )
    def k(idx_hbm, table_hbm, out_hbm, idx_v, g_v, w_v, g0, g1, w0, w1):
        wid = lax.axis_index("s") * nc + lax.axis_index("c")
        base = wid * per_w
        gsem = (g0, g1)
        wsem = (w0, w1)
        pltpu.sync_copy(idx_hbm.at[pl.ds(base, per_w)], idx_v)

        def issue_gather(c, b):
            pltpu.async_copy(
                table_hbm.at[idx_v.at[pl.ds(c * CHUNK, CHUNK)]],
                g_v.at[b],
                gsem[b],
            )

        def wait_gather(b):
            pltpu.make_async_copy(
                table_hbm.at[pl.ds(0, CHUNK)], g_v.at[b], gsem[b]
            ).wait()

        def issue_wb(c, b):
            pltpu.async_copy(
                w_v.at[b],
                out_hbm.at[pl.ds(base + c * CHUNK, CHUNK)],
                wsem[b],
            )

        def wait_wb(b):
            pltpu.make_async_copy(
                w_v.at[b],
                out_hbm.at[pl.ds(base, CHUNK)],
                wsem[b],
            ).wait()

        def compact(b):
            gb = g_v.at[b]
            wb = w_v.at[b]

            def rows(i, carry):
                for u in range(UNROLL):
                    r = i * UNROLL + u
                    for l in range(EMBED_DIM // VREG):
                        wb[r, pl.ds(l * VREG, VREG)] = gb[r, pl.ds(l * VREG, VREG)]
                return carry

            lax.fori_loop(0, CHUNK // UNROLL, rows, 0)

        issue_gather(0, 0)
        issue_gather(1, 1)

        def body(p, carry):
            for b in (0, 1):
                c = 2 * p + b
                wait_gather(b)

                @pl.when(p > 0)
                def _():
                    wait_wb(b)

                compact(b)

                @pl.when(p < npairs - 1)
                def _():
                    issue_gather(c + 2, b)

                issue_wb(c, b)
            return carry

        lax.fori_loop(0, npairs, body, 0)
        wait_wb(0)
        wait_wb(1)

    return k


def kernel(batch, table):
    k = _make_sc_gather()
    table128 = jnp.pad(table.T, ((0, LANES - EMBED_DIM), (0, 0))).T
    idx = batch.reshape(N)
    out = k(idx, table128)
    return out.reshape(BATCH, SEQ_LEN, EMBED_DIM)
